# baseline (speedup 1.0000x reference)
import functools

import jax
import jax.numpy as jnp
from jax import lax
from jax.experimental import pallas as pl
from jax.experimental.pallas import tpu as pltpu

_B, _C, _H, _W, _E = 2, 384, 48, 48, 128
_N = _H * _W
_SIGMA = 0.05
_SCALE = float(_E) ** (-0.5)
_NEG = float(jnp.finfo(jnp.float32).min)
_LOG2E = 1.4426950408889634
_CB = 200.0 * _LOG2E / ((_H - 1) * (_H - 1))

_BLK_KV = 768
_BLK_Q = 256
_BLK_K = 256
_NW = 32
_ROWS_PER_W = (_B * _N) // _NW


def _sc_gather(table, idx):
    from jax.experimental.pallas import tpu_sc as plsc
    mesh = plsc.VectorSubcoreMesh(core_axis_name="c", subcore_axis_name="s")

    @functools.partial(
        pl.kernel, mesh=mesh,
        out_type=jax.ShapeDtypeStruct((_B * _N, _C), jnp.float32),
        scratch_types=[
            pltpu.VMEM((_ROWS_PER_W,), jnp.int32),
            pltpu.VMEM((_ROWS_PER_W, _C), jnp.float32),
            pltpu.SemaphoreType.DMA,
        ],
    )
    def k(table_hbm, idx_hbm, out_hbm, idx_v, rows_v, sem):
        wid = lax.axis_index("s") * 2 + lax.axis_index("c")
        base = wid * _ROWS_PER_W
        pltpu.sync_copy(idx_hbm.at[pl.ds(base, _ROWS_PER_W)], idx_v)
        pltpu.async_copy(table_hbm.at[idx_v], rows_v, sem).wait()
        pltpu.sync_copy(rows_v, out_hbm.at[pl.ds(base, _ROWS_PER_W)])

    return k(table, idx)


def _kv_kernel(x_ref, w_ref, o_ref):
    o_ref[0] = jax.lax.dot_general(
        x_ref[0].astype(jnp.bfloat16), w_ref[...].astype(jnp.bfloat16),
        (((1,), (1,)), ((), ())),
        preferred_element_type=jnp.float32,
    )


def _attn_kernel(nkb_ref, x_ref, wq_ref, k_ref, v_ref, qi_ref, ki_ref,
                 col_ref, qcol_ref, hi_ref, wp_ref, o_ref):
    b = pl.program_id(0)
    q = (jax.lax.dot_general(
        wq_ref[...].astype(jnp.bfloat16), x_ref[0].astype(jnp.bfloat16),
        (((1,), (0,)), ((), ())),
        preferred_element_type=jnp.float32,
    ) * (_SCALE * _LOG2E)).astype(jnp.bfloat16)
    qi = qi_ref[...]
    qcol = qcol_ref[...]

    def body(kb, carry):
        acc, l = carry
        off = kb * _BLK_K
        k_blk = k_ref[0, pl.ds(off, _BLK_K), :].astype(jnp.bfloat16)
        v_blk = v_ref[0, pl.ds(off, _BLK_K), :].astype(jnp.bfloat16)
        s = jax.lax.dot_general(
            q, k_blk, (((0,), (1,)), ((), ())),
            preferred_element_type=jnp.float32,
        )
        cross = jax.lax.dot_general(
            qi, ki_ref[0, :, pl.ds(off, _BLK_K)], (((0,), (0,)), ((), ())),
            preferred_element_type=jnp.float32,
        )
        s = (s + qcol) + (cross * (2.0 * _CB) + col_ref[0, :, pl.ds(off, _BLK_K)])
        p = jnp.exp2(s)
        l = l + jnp.sum(p, axis=1, keepdims=True)
        acc = acc + jax.lax.dot_general(
            p.astype(jnp.bfloat16), v_blk, (((1,), (0,)), ((), ())),
            preferred_element_type=jnp.float32,
        )
        return acc, l

    acc, l = lax.fori_loop(
        0, nkb_ref[b], body,
        (jnp.zeros((_BLK_Q, _E), jnp.float32),
         jnp.zeros((_BLK_Q, 1), jnp.float32)))
    acc = (acc * (1.0 / jnp.maximum(l, 1e-30))).astype(jnp.bfloat16)
    delta_t = jax.lax.dot_general(
        wp_ref[...].astype(jnp.bfloat16), acc, (((1,), (1,)), ((), ())),
        preferred_element_type=jnp.float32,
    )
    o_ref[0] = x_ref[0] + jnp.where(hi_ref[0] > 0, delta_t, 0.0)


@jax.jit
def kernel(feat, mask_hi, Wq, Wk, Wv, Wp):
    x = feat.reshape(_B, _C, _N)
    wkv = jnp.concatenate([Wk, Wv], axis=0)
    hi_b = mask_hi.reshape(_B, _N)
    hi = hi_b.reshape(_B, 1, _N).astype(jnp.float32)

    cnt_lo = _N - jnp.sum(hi_b.astype(jnp.int32), axis=1)
    perm_lo = jnp.argsort(hi_b, axis=1, stable=True).astype(jnp.int32)
    nkb = (cnt_lo + _BLK_K - 1) // _BLK_K

    idx = jnp.arange(_N, dtype=jnp.int32)
    gi = (idx // _W).astype(jnp.float32)
    gj = (idx % _W).astype(jnp.float32)
    zero = jnp.zeros((_N,), jnp.float32)
    qcoords = jnp.stack([gi, gj, zero, zero, zero, zero, zero, zero],
                        axis=0).astype(jnp.bfloat16)
    kgi = gi[perm_lo]
    kgj = gj[perm_lo]
    zb = jnp.zeros((_B, _N), jnp.float32)
    kcoords = jnp.stack([kgi, kgj, zb, zb, zb, zb, zb, zb],
                        axis=1).astype(jnp.bfloat16)
    col = jnp.where(idx[None, :] < cnt_lo[:, None],
                    -_CB * (kgi * kgi + kgj * kgj),
                    _NEG).reshape(_B, 1, _N)
    qcol = (-_CB * (gi * gi + gj * gj)).reshape(_N, 1)

    ff_tm = x.transpose(0, 2, 1).reshape(_B * _N, _C)
    idx_flat = (perm_lo + (jnp.arange(_B, dtype=jnp.int32) * _N)[:, None]
                ).reshape(_B * _N)
    ffl = ff_tm[idx_flat].reshape(_B, _N, _C)

    kvc = pl.pallas_call(
        _kv_kernel,
        grid=(_B, _N // _BLK_KV),
        in_specs=[
            pl.BlockSpec((1, _BLK_KV, _C), lambda b, n: (b, n, 0)),
            pl.BlockSpec((2 * _E, _C), lambda b, n: (0, 0)),
        ],
        out_specs=pl.BlockSpec((1, _BLK_KV, 2 * _E), lambda b, n: (b, n, 0)),
        out_shape=jax.ShapeDtypeStruct((_B, _N, 2 * _E), jnp.float32),
    )(ffl, wkv)

    grid_spec = pltpu.PrefetchScalarGridSpec(
        num_scalar_prefetch=1,
        grid=(_B, _N // _BLK_Q),
        in_specs=[
            pl.BlockSpec((1, _C, _BLK_Q), lambda b, q, s: (b, 0, q)),
            pl.BlockSpec((_E, _C), lambda b, q, s: (0, 0)),
            pl.BlockSpec((1, _N, _E), lambda b, q, s: (b, 0, 0)),
            pl.BlockSpec((1, _N, _E), lambda b, q, s: (b, 0, 1)),
            pl.BlockSpec((8, _BLK_Q), lambda b, q, s: (0, q)),
            pl.BlockSpec((1, 8, _N), lambda b, q, s: (b, 0, 0)),
            pl.BlockSpec((1, 1, _N), lambda b, q, s: (b, 0, 0)),
            pl.BlockSpec((_BLK_Q, 1), lambda b, q, s: (q, 0)),
            pl.BlockSpec((1, 1, _BLK_Q), lambda b, q, s: (b, 0, q)),
            pl.BlockSpec((_C, _E), lambda b, q, s: (0, 0)),
        ],
        out_specs=pl.BlockSpec((1, _C, _BLK_Q), lambda b, q, s: (b, 0, q)),
    )

    out = pl.pallas_call(
        _attn_kernel,
        grid_spec=grid_spec,
        out_shape=jax.ShapeDtypeStruct((_B, _C, _N), jnp.float32),
    )(nkb, x, Wq, kvc, kvc, qcoords, kcoords, col, qcol, hi, Wp)

    return out.reshape(_B, _C, _H, _W)


# speedup vs baseline: 1.3851x; 1.3851x over previous
import functools

import jax
import jax.numpy as jnp
from jax import lax
from jax.experimental import pallas as pl
from jax.experimental.pallas import tpu as pltpu

_B, _C, _H, _W, _E = 2, 384, 48, 48, 128
_N = _H * _W
_SIGMA = 0.05
_SCALE = float(_E) ** (-0.5)
_NEG = float(jnp.finfo(jnp.float32).min)
_LOG2E = 1.4426950408889634
_CB = 200.0 * _LOG2E / ((_H - 1) * (_H - 1))

_BLK_K = 256
_NW = 32
_ROWS_PER_W = (_B * _N) // _NW


def _sc_gather(table, idx):
    from jax.experimental.pallas import tpu_sc as plsc
    mesh = plsc.VectorSubcoreMesh(core_axis_name="c", subcore_axis_name="s")

    @functools.partial(
        pl.kernel, mesh=mesh,
        out_type=jax.ShapeDtypeStruct((_B * _N, _C), jnp.float32),
        scratch_types=[
            pltpu.VMEM((_ROWS_PER_W,), jnp.int32),
            pltpu.VMEM((_ROWS_PER_W, _C), jnp.float32),
            pltpu.SemaphoreType.DMA,
        ],
    )
    def k(table_hbm, idx_hbm, out_hbm, idx_v, rows_v, sem):
        wid = lax.axis_index("s") * 2 + lax.axis_index("c")
        base = wid * _ROWS_PER_W
        pltpu.sync_copy(idx_hbm.at[pl.ds(base, _ROWS_PER_W)], idx_v)
        pltpu.async_copy(table_hbm.at[idx_v], rows_v, sem).wait()
        pltpu.sync_copy(rows_v, out_hbm.at[pl.ds(base, _ROWS_PER_W)])

    return k(table, idx)


def _attn_kernel(nkb_ref, x_ref, ffl_ref, wq_ref, wkv_ref, qi_ref, ki_ref,
                 col_ref, qcol_ref, hi_ref, wp_ref, o_ref):
    b = pl.program_id(0)
    q = (jax.lax.dot_general(
        wq_ref[...].astype(jnp.bfloat16), x_ref[0].astype(jnp.bfloat16),
        (((1,), (0,)), ((), ())),
        preferred_element_type=jnp.float32,
    ) * (_SCALE * _LOG2E)).astype(jnp.bfloat16)
    qi = qi_ref[...]
    qcol = qcol_ref[...]
    wkv = wkv_ref[...].astype(jnp.bfloat16)
    wp = wp_ref[...].astype(jnp.bfloat16)

    def body(kb, carry):
        acc, l = carry
        off = kb * _BLK_K
        kv = jax.lax.dot_general(
            ffl_ref[0, pl.ds(off, _BLK_K), :].astype(jnp.bfloat16), wkv,
            (((1,), (1,)), ((), ())),
            preferred_element_type=jnp.float32,
        )
        k_blk = kv[:, :_E].astype(jnp.bfloat16)
        v_blk = kv[:, _E:].astype(jnp.bfloat16)
        s = jax.lax.dot_general(
            q, k_blk, (((0,), (1,)), ((), ())),
            preferred_element_type=jnp.float32,
        )
        cross = jax.lax.dot_general(
            qi, ki_ref[0, :, pl.ds(off, _BLK_K)], (((0,), (0,)), ((), ())),
            preferred_element_type=jnp.float32,
        )
        s = (s + qcol) + (cross * (2.0 * _CB) + col_ref[0, :, pl.ds(off, _BLK_K)])
        p = jnp.exp2(s)
        l = l + jnp.sum(p, axis=1, keepdims=True)
        acc = acc + jax.lax.dot_general(
            p.astype(jnp.bfloat16), v_blk, (((1,), (0,)), ((), ())),
            preferred_element_type=jnp.float32,
        )
        return acc, l

    acc, l = lax.fori_loop(
        0, nkb_ref[b], body,
        (jnp.zeros((_N, _E), jnp.float32),
         jnp.zeros((_N, 1), jnp.float32)))
    acc = (acc * (1.0 / jnp.maximum(l, 1e-30))).astype(jnp.bfloat16)
    delta_t = jax.lax.dot_general(
        wp, acc, (((1,), (1,)), ((), ())),
        preferred_element_type=jnp.float32,
    )
    o_ref[0] = x_ref[0] + jnp.where(hi_ref[0] > 0, delta_t, 0.0)


@jax.jit
def kernel(feat, mask_hi, Wq, Wk, Wv, Wp):
    x = feat.reshape(_B, _C, _N)
    wkv = jnp.concatenate([Wk, Wv], axis=0)
    hi_b = mask_hi.reshape(_B, _N)
    hi = hi_b.reshape(_B, 1, _N).astype(jnp.float32)

    cnt_lo = _N - jnp.sum(hi_b.astype(jnp.int32), axis=1)
    perm_lo = jnp.argsort(hi_b, axis=1, stable=True).astype(jnp.int32)
    nkb = (cnt_lo + _BLK_K - 1) // _BLK_K

    idx = jnp.arange(_N, dtype=jnp.int32)
    gi = (idx // _W).astype(jnp.float32)
    gj = (idx % _W).astype(jnp.float32)
    zero = jnp.zeros((_N,), jnp.float32)
    qcoords = jnp.stack([gi, gj, zero, zero, zero, zero, zero, zero],
                        axis=0).astype(jnp.bfloat16)
    kgi = gi[perm_lo]
    kgj = gj[perm_lo]
    zb = jnp.zeros((_B, _N), jnp.float32)
    kcoords = jnp.stack([kgi, kgj, zb, zb, zb, zb, zb, zb],
                        axis=1).astype(jnp.bfloat16)
    col = jnp.where(idx[None, :] < cnt_lo[:, None],
                    -_CB * (kgi * kgi + kgj * kgj),
                    _NEG).reshape(_B, 1, _N)
    qcol = (-_CB * (gi * gi + gj * gj)).reshape(_N, 1)

    ff_tm = x.transpose(0, 2, 1).reshape(_B * _N, _C)
    idx_flat = (perm_lo + (jnp.arange(_B, dtype=jnp.int32) * _N)[:, None]
                ).reshape(_B * _N)
    ffl = _sc_gather(ff_tm, idx_flat).reshape(_B, _N, _C)

    grid_spec = pltpu.PrefetchScalarGridSpec(
        num_scalar_prefetch=1,
        grid=(_B,),
        in_specs=[
            pl.BlockSpec((1, _C, _N), lambda b, s: (b, 0, 0)),
            pl.BlockSpec((1, _N, _C), lambda b, s: (b, 0, 0)),
            pl.BlockSpec((_E, _C), lambda b, s: (0, 0)),
            pl.BlockSpec((2 * _E, _C), lambda b, s: (0, 0)),
            pl.BlockSpec((8, _N), lambda b, s: (0, 0)),
            pl.BlockSpec((1, 8, _N), lambda b, s: (b, 0, 0)),
            pl.BlockSpec((1, 1, _N), lambda b, s: (b, 0, 0)),
            pl.BlockSpec((_N, 1), lambda b, s: (0, 0)),
            pl.BlockSpec((1, 1, _N), lambda b, s: (b, 0, 0)),
            pl.BlockSpec((_C, _E), lambda b, s: (0, 0)),
        ],
        out_specs=pl.BlockSpec((1, _C, _N), lambda b, s: (b, 0, 0)),
    )

    out = pl.pallas_call(
        _attn_kernel,
        grid_spec=grid_spec,
        out_shape=jax.ShapeDtypeStruct((_B, _C, _N), jnp.float32),
    )(nkb, x, ffl, Wq, wkv, qcoords, kcoords, col, qcol, hi, Wp)

    return out.reshape(_B, _C, _H, _W)


# speedup vs baseline: 1.5987x; 1.1542x over previous
import functools

import jax
import jax.numpy as jnp
from jax import lax
from jax.experimental import pallas as pl
from jax.experimental.pallas import tpu as pltpu

_B, _C, _H, _W, _E = 2, 384, 48, 48, 128
_N = _H * _W
_SIGMA = 0.05
_SCALE = float(_E) ** (-0.5)
_NEG = float(jnp.finfo(jnp.float32).min)
_LOG2E = 1.4426950408889634
_CB = 200.0 * _LOG2E / ((_H - 1) * (_H - 1))

_BLK_K = 256
_NW = 32
_ROWS_PER_W = (_B * _N) // _NW


def _sc_gather(table, idx):
    from jax.experimental.pallas import tpu_sc as plsc
    mesh = plsc.VectorSubcoreMesh(core_axis_name="c", subcore_axis_name="s")

    @functools.partial(
        pl.kernel, mesh=mesh,
        out_type=jax.ShapeDtypeStruct((_B * _N, _C), jnp.float32),
        scratch_types=[
            pltpu.VMEM((_ROWS_PER_W,), jnp.int32),
            pltpu.VMEM((_ROWS_PER_W, _C), jnp.float32),
            pltpu.SemaphoreType.DMA,
        ],
    )
    def k(table_hbm, idx_hbm, out_hbm, idx_v, rows_v, sem):
        wid = lax.axis_index("s") * 2 + lax.axis_index("c")
        base = wid * _ROWS_PER_W
        pltpu.sync_copy(idx_hbm.at[pl.ds(base, _ROWS_PER_W)], idx_v)
        pltpu.async_copy(table_hbm.at[idx_v], rows_v, sem).wait()
        pltpu.sync_copy(rows_v, out_hbm.at[pl.ds(base, _ROWS_PER_W)])

    return k(table, idx)


def _attn_kernel(nkb_ref, x_ref, ffl_ref, wq_ref, wkv_ref, qi_ref, ki_ref,
                 col_ref, qcol_ref, hi_ref, wp_ref, o_ref):
    b = pl.program_id(0)
    q = (jax.lax.dot_general(
        wq_ref[...].astype(jnp.bfloat16), x_ref[0].astype(jnp.bfloat16),
        (((1,), (0,)), ((), ())),
        preferred_element_type=jnp.float32,
    ) * (_SCALE * _LOG2E)).astype(jnp.bfloat16)
    qi = qi_ref[...]
    qcol = qcol_ref[...]
    wkv = wkv_ref[...].astype(jnp.bfloat16)
    wp = wp_ref[...].astype(jnp.bfloat16)

    def body(kb, carry):
        acc, l = carry
        off = kb * _BLK_K
        kv = jax.lax.dot_general(
            ffl_ref[0, pl.ds(off, _BLK_K), :].astype(jnp.bfloat16), wkv,
            (((1,), (1,)), ((), ())),
            preferred_element_type=jnp.float32,
        )
        k_blk = kv[:, :_E].astype(jnp.bfloat16)
        v_blk = kv[:, _E:].astype(jnp.bfloat16)
        s = jax.lax.dot_general(
            q, k_blk, (((0,), (1,)), ((), ())),
            preferred_element_type=jnp.float32,
        )
        cross = jax.lax.dot_general(
            qi, ki_ref[0, :, pl.ds(off, _BLK_K)], (((0,), (0,)), ((), ())),
            preferred_element_type=jnp.float32,
        )
        s = (s + qcol) + (cross * (2.0 * _CB) + col_ref[0, :, pl.ds(off, _BLK_K)])
        p = jnp.exp2(s)
        l = l + jnp.sum(p, axis=1, keepdims=True)
        acc = acc + jax.lax.dot_general(
            p.astype(jnp.bfloat16), v_blk, (((1,), (0,)), ((), ())),
            preferred_element_type=jnp.float32,
        )
        return acc, l

    acc, l = lax.fori_loop(
        0, nkb_ref[b], body,
        (jnp.zeros((_N, _E), jnp.float32),
         jnp.zeros((_N, 1), jnp.float32)))
    acc = (acc * (1.0 / jnp.maximum(l, 1e-30))).astype(jnp.bfloat16)
    delta_t = jax.lax.dot_general(
        wp, acc, (((1,), (1,)), ((), ())),
        preferred_element_type=jnp.float32,
    )
    o_ref[0] = x_ref[0] + jnp.where(hi_ref[0] > 0, delta_t, 0.0)


@jax.jit
def kernel(feat, mask_hi, Wq, Wk, Wv, Wp):
    x = feat.reshape(_B, _C, _N)
    wkv = jnp.concatenate([Wk, Wv], axis=0)
    hi_b = mask_hi.reshape(_B, _N)
    hi = hi_b.reshape(_B, 1, _N).astype(jnp.float32)

    cnt_lo = _N - jnp.sum(hi_b.astype(jnp.int32), axis=1)
    perm_lo = jnp.argsort(hi_b, axis=1, stable=True).astype(jnp.int32)
    nkb = jnp.zeros_like(cnt_lo)

    idx = jnp.arange(_N, dtype=jnp.int32)
    gi = (idx // _W).astype(jnp.float32)
    gj = (idx % _W).astype(jnp.float32)
    zero = jnp.zeros((_N,), jnp.float32)
    qcoords = jnp.stack([gi, gj, zero, zero, zero, zero, zero, zero],
                        axis=0).astype(jnp.bfloat16)
    kgi = gi[perm_lo]
    kgj = gj[perm_lo]
    zb = jnp.zeros((_B, _N), jnp.float32)
    kcoords = jnp.stack([kgi, kgj, zb, zb, zb, zb, zb, zb],
                        axis=1).astype(jnp.bfloat16)
    col = jnp.where(idx[None, :] < cnt_lo[:, None],
                    -_CB * (kgi * kgi + kgj * kgj),
                    _NEG).reshape(_B, 1, _N)
    qcol = (-_CB * (gi * gi + gj * gj)).reshape(_N, 1)

    ff_tm = x.transpose(0, 2, 1).reshape(_B * _N, _C)
    idx_flat = (perm_lo + (jnp.arange(_B, dtype=jnp.int32) * _N)[:, None]
                ).reshape(_B * _N)
    ffl = _sc_gather(ff_tm, idx_flat).reshape(_B, _N, _C)

    grid_spec = pltpu.PrefetchScalarGridSpec(
        num_scalar_prefetch=1,
        grid=(_B,),
        in_specs=[
            pl.BlockSpec((1, _C, _N), lambda b, s: (b, 0, 0)),
            pl.BlockSpec((1, _N, _C), lambda b, s: (b, 0, 0)),
            pl.BlockSpec((_E, _C), lambda b, s: (0, 0)),
            pl.BlockSpec((2 * _E, _C), lambda b, s: (0, 0)),
            pl.BlockSpec((8, _N), lambda b, s: (0, 0)),
            pl.BlockSpec((1, 8, _N), lambda b, s: (b, 0, 0)),
            pl.BlockSpec((1, 1, _N), lambda b, s: (b, 0, 0)),
            pl.BlockSpec((_N, 1), lambda b, s: (0, 0)),
            pl.BlockSpec((1, 1, _N), lambda b, s: (b, 0, 0)),
            pl.BlockSpec((_C, _E), lambda b, s: (0, 0)),
        ],
        out_specs=pl.BlockSpec((1, _C, _N), lambda b, s: (b, 0, 0)),
    )

    out = pl.pallas_call(
        _attn_kernel,
        grid_spec=grid_spec,
        out_shape=jax.ShapeDtypeStruct((_B, _C, _N), jnp.float32),
    )(nkb, x, ffl, Wq, wkv, qcoords, kcoords, col, qcol, hi, Wp)

    return out.reshape(_B, _C, _H, _W)


# speedup vs baseline: 3.3545x; 2.0983x over previous
import jax
import jax.numpy as jnp
from jax import lax
from jax.experimental import pallas as pl
from jax.experimental.pallas import tpu as pltpu

_B, _C, _H, _W, _E = 2, 384, 48, 48, 128
_N = _H * _W
_SIGMA = 0.05
_SCALE = float(_E) ** (-0.5)
_NEG = float(jnp.finfo(jnp.float32).min)
_LOG2E = 1.4426950408889634
_CB = 200.0 * _LOG2E / ((_H - 1) * (_H - 1))

_BLK_Q = 768
_NQ = _N // _BLK_Q


def _coords_i(idx_i32):
    gi = (idx_i32 // _W).astype(jnp.float32)
    gj = (idx_i32 % _W).astype(jnp.float32)
    return gi, gj


def _attn_kernel(x_blk_ref, x_full_ref, m_ref, wq_ref, wk_ref, wv_ref,
                 wp_ref, o_ref, k_s, v_s):
    qb = pl.program_id(1)

    @pl.when(qb == 0)
    def _project_kv():
        xf = x_full_ref[0].astype(jnp.bfloat16)
        k_s[...] = jax.lax.dot_general(
            xf, wk_ref[...].astype(jnp.bfloat16),
            (((0,), (1,)), ((), ())),
            preferred_element_type=jnp.float32,
        ).astype(jnp.bfloat16)
        v_s[...] = jax.lax.dot_general(
            xf, wv_ref[...].astype(jnp.bfloat16),
            (((0,), (1,)), ((), ())),
            preferred_element_type=jnp.float32,
        ).astype(jnp.bfloat16)

    q = (jax.lax.dot_general(
        wq_ref[...].astype(jnp.bfloat16), x_blk_ref[0].astype(jnp.bfloat16),
        (((1,), (0,)), ((), ())),
        preferred_element_type=jnp.float32,
    ) * (_SCALE * _LOG2E)).astype(jnp.bfloat16)

    qidx = qb * _BLK_Q + jax.lax.broadcasted_iota(jnp.int32, (1, _BLK_Q), 1)
    qgi, qgj = _coords_i(qidx)
    qi2 = jnp.concatenate([qgi, qgj], axis=0).astype(jnp.bfloat16)
    kidx = jax.lax.broadcasted_iota(jnp.int32, (1, _N), 1)
    kgi, kgj = _coords_i(kidx)
    ki2 = jnp.concatenate([kgi, kgj], axis=0).astype(jnp.bfloat16)

    qcol = (-_CB) * (qgi * qgi + qgj * qgj)
    qcol = qcol.reshape(_BLK_Q, 1)
    col = jnp.where(m_ref[0] > 0, _NEG,
                    (-_CB) * (kgi * kgi + kgj * kgj))

    s = jax.lax.dot_general(
        q, k_s[...], (((0,), (1,)), ((), ())),
        preferred_element_type=jnp.float32,
    )
    cross = jax.lax.dot_general(
        qi2, ki2, (((0,), (0,)), ((), ())),
        preferred_element_type=jnp.float32,
    )
    s = (s + qcol) + (cross * (2.0 * _CB) + col)
    p = jnp.exp2(s)
    l = jnp.sum(p, axis=1, keepdims=True)
    agg = jax.lax.dot_general(
        p.astype(jnp.bfloat16), v_s[...], (((1,), (0,)), ((), ())),
        preferred_element_type=jnp.float32,
    )
    agg = (agg * (1.0 / jnp.maximum(l, 1e-30))).astype(jnp.bfloat16)
    delta_t = jax.lax.dot_general(
        wp_ref[...].astype(jnp.bfloat16), agg, (((1,), (1,)), ((), ())),
        preferred_element_type=jnp.float32,
    )
    m_blk = m_ref[0, :, pl.ds(qb * _BLK_Q, _BLK_Q)]
    o_ref[0] = x_blk_ref[0] + jnp.where(m_blk > 0, delta_t, 0.0)


@jax.jit
def kernel(feat, mask_hi, Wq, Wk, Wv, Wp):
    x = feat.reshape(_B, _C, _N)
    m = mask_hi.reshape(_B, 1, _N)

    out = pl.pallas_call(
        _attn_kernel,
        grid=(_B, _NQ),
        in_specs=[
            pl.BlockSpec((1, _C, _BLK_Q), lambda b, q: (b, 0, q)),
            pl.BlockSpec((1, _C, _N), lambda b, q: (b, 0, 0)),
            pl.BlockSpec((1, 1, _N), lambda b, q: (b, 0, 0)),
            pl.BlockSpec((_E, _C), lambda b, q: (0, 0)),
            pl.BlockSpec((_E, _C), lambda b, q: (0, 0)),
            pl.BlockSpec((_E, _C), lambda b, q: (0, 0)),
            pl.BlockSpec((_C, _E), lambda b, q: (0, 0)),
        ],
        out_specs=pl.BlockSpec((1, _C, _BLK_Q), lambda b, q: (b, 0, q)),
        out_shape=jax.ShapeDtypeStruct((_B, _C, _N), jnp.float32),
        scratch_shapes=[
            pltpu.VMEM((_N, _E), jnp.bfloat16),
            pltpu.VMEM((_N, _E), jnp.bfloat16),
        ],
    )(x, x, m, Wq, Wk, Wv, Wp)

    return out.reshape(_B, _C, _H, _W)


# speedup vs baseline: 3.4132x; 1.0175x over previous
import jax
import jax.numpy as jnp
from jax import lax
from jax.experimental import pallas as pl
from jax.experimental.pallas import tpu as pltpu

_B, _C, _H, _W, _E = 2, 384, 48, 48, 128
_N = _H * _W
_SIGMA = 0.05
_SCALE = float(_E) ** (-0.5)
_NEG = float(jnp.finfo(jnp.float32).min)
_LOG2E = 1.4426950408889634
_CB = 200.0 * _LOG2E / ((_H - 1) * (_H - 1))

_BLK_Q = 1152
_NQ = _N // _BLK_Q


def _coords_i(idx_i32):
    gi = (idx_i32 // _W).astype(jnp.float32)
    gj = (idx_i32 % _W).astype(jnp.float32)
    return gi, gj


def _attn_kernel(x_blk_ref, x_full_ref, m_ref, wq_ref, wk_ref, wv_ref,
                 wp_ref, o_ref, k_s, v_s):
    qb = pl.program_id(1)

    @pl.when(qb == 0)
    def _project_kv():
        xf = x_full_ref[0].astype(jnp.bfloat16)
        k_s[...] = jax.lax.dot_general(
            xf, wk_ref[...].astype(jnp.bfloat16),
            (((0,), (1,)), ((), ())),
            preferred_element_type=jnp.float32,
        ).astype(jnp.bfloat16)
        v_s[...] = jax.lax.dot_general(
            xf, wv_ref[...].astype(jnp.bfloat16),
            (((0,), (1,)), ((), ())),
            preferred_element_type=jnp.float32,
        ).astype(jnp.bfloat16)

    q = (jax.lax.dot_general(
        wq_ref[...].astype(jnp.bfloat16), x_blk_ref[0].astype(jnp.bfloat16),
        (((1,), (0,)), ((), ())),
        preferred_element_type=jnp.float32,
    ) * (_SCALE * _LOG2E)).astype(jnp.bfloat16)

    qidx = qb * _BLK_Q + jax.lax.broadcasted_iota(jnp.int32, (1, _BLK_Q), 1)
    qgi, qgj = _coords_i(qidx)
    qc3 = -0.5 * (qgi * qgi + qgj * qgj)
    qi2 = jnp.concatenate([qgi, qgj, qc3], axis=0).astype(jnp.bfloat16)
    kidx = jax.lax.broadcasted_iota(jnp.int32, (1, _N), 1)
    kgi, kgj = _coords_i(kidx)
    ones = jnp.ones((1, _N), jnp.float32)
    ki2 = jnp.concatenate([kgi, kgj, ones], axis=0).astype(jnp.bfloat16)

    col = jnp.where(m_ref[0] > 0, _NEG,
                    (-_CB) * (kgi * kgi + kgj * kgj))

    s = jax.lax.dot_general(
        q, k_s[...], (((0,), (1,)), ((), ())),
        preferred_element_type=jnp.float32,
    )
    cross = jax.lax.dot_general(
        qi2, ki2, (((0,), (0,)), ((), ())),
        preferred_element_type=jnp.float32,
    )
    s = s + (cross * (2.0 * _CB) + col)
    p = jnp.exp2(s)
    l = jnp.sum(p, axis=1, keepdims=True)
    agg = jax.lax.dot_general(
        p.astype(jnp.bfloat16), v_s[...], (((1,), (0,)), ((), ())),
        preferred_element_type=jnp.float32,
    )
    agg = (agg * (1.0 / jnp.maximum(l, 1e-30))).astype(jnp.bfloat16)
    delta_t = jax.lax.dot_general(
        wp_ref[...].astype(jnp.bfloat16), agg, (((1,), (1,)), ((), ())),
        preferred_element_type=jnp.float32,
    )
    m_blk = m_ref[0, :, pl.ds(qb * _BLK_Q, _BLK_Q)]
    o_ref[0] = x_blk_ref[0] + jnp.where(m_blk > 0, delta_t, 0.0)


@jax.jit
def kernel(feat, mask_hi, Wq, Wk, Wv, Wp):
    x = feat.reshape(_B, _C, _N)
    m = mask_hi.reshape(_B, 1, _N)

    out = pl.pallas_call(
        _attn_kernel,
        grid=(_B, _NQ),
        in_specs=[
            pl.BlockSpec((1, _C, _BLK_Q), lambda b, q: (b, 0, q)),
            pl.BlockSpec((1, _C, _N), lambda b, q: (b, 0, 0)),
            pl.BlockSpec((1, 1, _N), lambda b, q: (b, 0, 0)),
            pl.BlockSpec((_E, _C), lambda b, q: (0, 0)),
            pl.BlockSpec((_E, _C), lambda b, q: (0, 0)),
            pl.BlockSpec((_E, _C), lambda b, q: (0, 0)),
            pl.BlockSpec((_C, _E), lambda b, q: (0, 0)),
        ],
        out_specs=pl.BlockSpec((1, _C, _BLK_Q), lambda b, q: (b, 0, q)),
        out_shape=jax.ShapeDtypeStruct((_B, _C, _N), jnp.float32),
        scratch_shapes=[
            pltpu.VMEM((_N, _E), jnp.bfloat16),
            pltpu.VMEM((_N, _E), jnp.bfloat16),
        ],
    )(x, x, m, Wq, Wk, Wv, Wp)

    return out.reshape(_B, _C, _H, _W)
